# baseline (speedup 1.0000x reference)
import jax
import jax.numpy as jnp
from jax.experimental import pallas as pl

B, N_H, N_PTS, K_SEL, NEIGHS = 2, 64, 4096, 1024, 20
EPS = 1e-5
HI = jax.lax.Precision.HIGHEST


def _copy_kernel(x_ref, o_ref):
    o_ref[...] = x_ref[...]


def _pl_copy(x):
    return pl.pallas_call(
        _copy_kernel,
        out_shape=jax.ShapeDtypeStruct(x.shape, x.dtype),
    )(x)


def kernel(xyz, seq1, W_fc, b_fc, g1, beta1, W_conv, g2, beta2, W_bil, b_bil, perm):
    A = jnp.einsum('oc,bcn->bon', W_fc, seq1) + b_fc[None, :, None]
    m1 = jnp.mean(A, axis=(0, 2), keepdims=True)
    v1 = jnp.var(A, axis=(0, 2), keepdims=True)
    h1 = jnp.maximum((A - m1) / jnp.sqrt(v1 + EPS) * g1.reshape(1, N_H, 1)
                     + beta1.reshape(1, N_H, 1), 0.0)

    xx = jnp.sum(h1 * h1, axis=1)
    G = jnp.einsum('bcn,bcm->bnm', h1, h1)
    pd = 2.0 * G - xx[:, :, None] - xx[:, None, :]
    idx20 = jax.lax.top_k(pd, NEIGHS)[1]

    h1t = jnp.transpose(h1, (0, 2, 1))
    feat = jax.vmap(lambda t, i: t[i])(h1t, idx20)
    ctr = h1t[:, :, None, :]
    gf = jnp.concatenate([feat - ctr, jnp.broadcast_to(ctr, feat.shape)], axis=3)
    gf = jnp.transpose(gf, (0, 3, 1, 2))
    hc = jnp.einsum('oc,bcnk->bonk', W_conv, gf)

    mean2 = jnp.mean(hc, axis=(0, 2, 3))
    e2 = jnp.mean(hc * hc, axis=(0, 2, 3))
    var2 = e2 - mean2 * mean2
    mx = jnp.max(hc, axis=3)
    hbn = ((mx - mean2[None, :, None]) / jnp.sqrt(var2[None, :, None] + EPS)
           * g2.reshape(1, N_H, 1) + beta2.reshape(1, N_H, 1))
    h_n1 = jnp.where(hbn > 0, hbn, 0.2 * hbn)
    X = jax.nn.sigmoid(h_n1)
    Xt = jnp.transpose(X, (0, 2, 1))

    T1 = jnp.einsum('bni,ij->bnj', h1t, W_bil)
    sc1 = jnp.sum(T1 * Xt, axis=2) + b_bil
    h2t = h1t[:, perm, :]
    T2 = jnp.einsum('bni,ij->bnj', h2t, W_bil)
    sc2 = jnp.sum(T2 * Xt, axis=2) + b_bil
    logits = jnp.concatenate([sc1, sc2], axis=1)

    scores = jax.nn.sigmoid(sc1)
    gt = (scores[:, None, :] > scores[:, :, None]).astype(jnp.int32)
    iot = jnp.arange(N_PTS)
    tie = ((scores[:, None, :] == scores[:, :, None])
           & (iot[None, None, :] < iot[None, :, None])).astype(jnp.int32)
    rank = jnp.sum(gt + tie, axis=2)
    M = (rank[:, None, :] == jnp.arange(K_SEL)[None, :, None]).astype(jnp.float32)
    values = jnp.einsum('bri,bi->br', M, scores, precision=HI)
    idx = jnp.einsum('bri,i->br', M, iot.astype(jnp.float32), precision=HI).astype(jnp.int32)
    seq_static = jnp.einsum('bcn,brn->bcr', seq1, M, precision=HI)
    seq = seq_static * values[:, None, :]
    xyz_static = jnp.einsum('bcn,brn->bcr', xyz, M, precision=HI)
    xyz_out = xyz_static * values[:, None, :]

    seq = _pl_copy(seq)
    return seq, values, idx, logits, xyz_static, xyz_out


# speedup vs baseline: 2.6923x; 2.6923x over previous
import functools
import jax
import jax.numpy as jnp
from jax.experimental import pallas as pl

B, N_H, N_PTS, K_SEL, NEIGHS = 2, 64, 4096, 1024, 20
EPS = 1e-5
HI = jax.lax.Precision.HIGHEST
RBLK = 256
NBLK = N_PTS // RBLK


def _knn_body(h1_ref, xx_ref, out_ref):
    i = pl.program_id(1)
    hb = h1_ref[0].astype(jnp.bfloat16)
    rows = h1_ref[0, :, pl.ds(i * RBLK, RBLK)].astype(jnp.bfloat16)
    A = jax.lax.dot_general(rows, hb, (((0,), (0,)), ((), ())),
                            preferred_element_type=jnp.float32)
    xx = xx_ref[0]
    xr = jnp.reshape(xx_ref[0, 0, pl.ds(i * RBLK, RBLK)], (RBLK, 1))
    P = 2.0 * A - xr - xx
    iota_j = jax.lax.broadcasted_iota(jnp.int32, (RBLK, N_PTS), 1)
    for k in range(NEIGHS):
        m = jnp.max(P, axis=1, keepdims=True)
        cand = jnp.where(P == m, iota_j, N_PTS)
        am = jnp.min(cand, axis=1)
        out_ref[0, k, :] = am
        P = jnp.where(cand == am[:, None], -jnp.inf, P)


def _knn_topk(h1, xx):
    return pl.pallas_call(
        _knn_body,
        grid=(B, NBLK),
        in_specs=[
            pl.BlockSpec((1, N_H, N_PTS), lambda b, i: (b, 0, 0)),
            pl.BlockSpec((1, 1, N_PTS), lambda b, i: (b, 0, 0)),
        ],
        out_specs=pl.BlockSpec((1, NEIGHS, RBLK), lambda b, i: (b, 0, i)),
        out_shape=jax.ShapeDtypeStruct((B, NEIGHS, N_PTS), jnp.int32),
    )(h1, xx.reshape(B, 1, N_PTS))


def kernel(xyz, seq1, W_fc, b_fc, g1, beta1, W_conv, g2, beta2, W_bil, b_bil, perm):
    A = jnp.einsum('oc,bcn->bon', W_fc, seq1) + b_fc[None, :, None]
    m1 = jnp.mean(A, axis=(0, 2), keepdims=True)
    v1 = jnp.var(A, axis=(0, 2), keepdims=True)
    h1 = jnp.maximum((A - m1) / jnp.sqrt(v1 + EPS) * g1.reshape(1, N_H, 1)
                     + beta1.reshape(1, N_H, 1), 0.0)

    xx = jnp.sum(h1 * h1, axis=1)
    idx20 = jnp.transpose(_knn_topk(h1, xx), (0, 2, 1))

    h1t = jnp.transpose(h1, (0, 2, 1))
    feat = jax.vmap(lambda t, i: t[i])(h1t, idx20)
    ctr = h1t[:, :, None, :]
    gf = jnp.concatenate([feat - ctr, jnp.broadcast_to(ctr, feat.shape)], axis=3)
    gf = jnp.transpose(gf, (0, 3, 1, 2))
    hc = jnp.einsum('oc,bcnk->bonk', W_conv, gf)

    mean2 = jnp.mean(hc, axis=(0, 2, 3))
    e2 = jnp.mean(hc * hc, axis=(0, 2, 3))
    var2 = e2 - mean2 * mean2
    mx = jnp.max(hc, axis=3)
    hbn = ((mx - mean2[None, :, None]) / jnp.sqrt(var2[None, :, None] + EPS)
           * g2.reshape(1, N_H, 1) + beta2.reshape(1, N_H, 1))
    h_n1 = jnp.where(hbn > 0, hbn, 0.2 * hbn)
    X = jax.nn.sigmoid(h_n1)
    Xt = jnp.transpose(X, (0, 2, 1))

    T1 = jnp.einsum('bni,ij->bnj', h1t, W_bil)
    sc1 = jnp.sum(T1 * Xt, axis=2) + b_bil
    h2t = h1t[:, perm, :]
    T2 = jnp.einsum('bni,ij->bnj', h2t, W_bil)
    sc2 = jnp.sum(T2 * Xt, axis=2) + b_bil
    logits = jnp.concatenate([sc1, sc2], axis=1)

    scores = jax.nn.sigmoid(sc1)
    gt = (scores[:, None, :] > scores[:, :, None]).astype(jnp.int32)
    iot = jnp.arange(N_PTS)
    tie = ((scores[:, None, :] == scores[:, :, None])
           & (iot[None, None, :] < iot[None, :, None])).astype(jnp.int32)
    rank = jnp.sum(gt + tie, axis=2)
    M = (rank[:, None, :] == jnp.arange(K_SEL)[None, :, None]).astype(jnp.float32)
    values = jnp.einsum('bri,bi->br', M, scores, precision=HI)
    idx = jnp.einsum('bri,i->br', M, iot.astype(jnp.float32), precision=HI).astype(jnp.int32)
    seq_static = jnp.einsum('bcn,brn->bcr', seq1, M, precision=HI)
    seq = seq_static * values[:, None, :]
    xyz_static = jnp.einsum('bcn,brn->bcr', xyz, M, precision=HI)
    xyz_out = xyz_static * values[:, None, :]

    return seq, values, idx, logits, xyz_static, xyz_out
